# baseline (speedup 1.0000x reference)
import functools
import math

import jax
import jax.numpy as jnp
from jax.experimental import pallas as pl

B, S, D = 1, 2048, 2048
H, KVH, HD = 16, 4, 128
E, K, F = 8, 2, 768
EPS = 1e-06
THETA = 10000.0

BT = 256
QKV_CB = 512
FB = 256


def _qkv_kernel(x_ref, w_ref, ln_ref, q_ref):
    x = x_ref[...]
    ms = jnp.mean(x * x, axis=-1, keepdims=True)
    h = x * jax.lax.rsqrt(ms + EPS) * ln_ref[...]
    q_ref[...] = jnp.dot(h, w_ref[...], preferred_element_type=jnp.float32)


def _norm_rope_kernel(x_ref, w_ref, o_ref):
    tb = pl.program_id(1)
    x = x_ref[...]
    ms = jnp.mean(x * x, axis=-1, keepdims=True)
    x = x * jax.lax.rsqrt(ms + EPS) * w_ref[...]
    half = HD // 2
    j = jax.lax.broadcasted_iota(jnp.int32, (BT, half), 1).astype(jnp.float32)
    inv = jnp.exp(j * (-math.log(THETA) / half))
    pos = (tb * BT + jax.lax.broadcasted_iota(jnp.int32, (BT, half), 0)).astype(jnp.float32)
    f = pos * inv
    cos = jnp.cos(f)
    sin = jnp.sin(f)
    x1 = x[:, :half]
    x2 = x[:, half:]
    o_ref[...] = jnp.concatenate([x1 * cos - x2 * sin, x2 * cos + x1 * sin], axis=-1)


def _attn_kernel(q_ref, k_ref, v_ref, o_ref):
    qb = pl.program_id(1)
    q = q_ref[...]
    k = k_ref[...]
    s = jax.lax.dot_general(q, k, (((1,), (1,)), ((), ())),
                            preferred_element_type=jnp.float32)
    s = s * (1.0 / math.sqrt(HD))
    row = qb * BT + jax.lax.broadcasted_iota(jnp.int32, (BT, S), 0)
    col = jax.lax.broadcasted_iota(jnp.int32, (BT, S), 1)
    s = jnp.where(col <= row, s, -1e30)
    m = jnp.max(s, axis=-1, keepdims=True)
    p = jnp.exp(s - m)
    l = jnp.sum(p, axis=-1, keepdims=True)
    o = jnp.dot(p, v_ref[...], preferred_element_type=jnp.float32)
    o_ref[...] = o / l


def _oproj_kernel(o_ref, w_ref, res_ref, x_ref):
    x_ref[...] = res_ref[...] + jnp.dot(o_ref[...], w_ref[...],
                                        preferred_element_type=jnp.float32)


def _router_kernel(x_ref, ln_ref, rw_ref, h_ref, coef_ref):
    x = x_ref[...]
    ms = jnp.mean(x * x, axis=-1, keepdims=True)
    h = x * jax.lax.rsqrt(ms + EPS) * ln_ref[...]
    h_ref[...] = h
    logits = jnp.dot(h, rw_ref[...], preferred_element_type=jnp.float32)
    m = jnp.max(logits, axis=-1, keepdims=True)
    p = jnp.exp(logits - m)
    probs = p / jnp.sum(p, axis=-1, keepdims=True)
    idx = jax.lax.broadcasted_iota(jnp.int32, (BT, E), 1)
    m1 = jnp.max(probs, axis=-1, keepdims=True)
    i1 = jnp.min(jnp.where(probs == m1, idx, E), axis=-1, keepdims=True)
    probs2 = jnp.where(idx == i1, -1.0, probs)
    m2 = jnp.max(probs2, axis=-1, keepdims=True)
    i2 = jnp.min(jnp.where(probs2 == m2, idx, E), axis=-1, keepdims=True)
    tot = m1 + m2
    w1 = m1 / tot
    w2 = m2 / tot
    coef_ref[...] = jnp.where(idx == i1, w1, 0.0) + jnp.where(idx == i2, w2, 0.0)


def _moe_kernel(h_ref, wg_ref, wu_ref, wd_ref, coef_ref, res_ref, o_ref):
    e = pl.program_id(1)
    fb = pl.program_id(2)
    h = h_ref[...]
    g = jnp.dot(h, wg_ref[0], preferred_element_type=jnp.float32)
    u = jnp.dot(h, wu_ref[0], preferred_element_type=jnp.float32)
    a = (g / (1.0 + jnp.exp(-g))) * u
    y = jnp.dot(a, wd_ref[0], preferred_element_type=jnp.float32)
    eidx = jax.lax.broadcasted_iota(jnp.int32, (BT, E), 1)
    c = jnp.sum(jnp.where(eidx == e, coef_ref[...], 0.0), axis=1, keepdims=True)
    y = c * y

    @pl.when(jnp.logical_and(e == 0, fb == 0))
    def _init():
        o_ref[...] = res_ref[...] + y

    @pl.when(jnp.logical_not(jnp.logical_and(e == 0, fb == 0)))
    def _acc():
        o_ref[...] = o_ref[...] + y


def kernel(hidden_states, ln1_w, Wq, Wk, Wv, q_norm_w, k_norm_w, Wo, ln2_w,
           router_W, W_gate, W_up, W_down):
    x = hidden_states.reshape(S, D)
    nt = S // BT

    Wqkv = jnp.concatenate([Wq, Wk, Wv], axis=1)
    QKVW = Wqkv.shape[1]
    qkv = pl.pallas_call(
        _qkv_kernel,
        grid=(nt, QKVW // QKV_CB),
        in_specs=[
            pl.BlockSpec((BT, D), lambda t, c: (t, 0)),
            pl.BlockSpec((D, QKV_CB), lambda t, c: (0, c)),
            pl.BlockSpec((1, D), lambda t, c: (0, 0)),
        ],
        out_specs=pl.BlockSpec((BT, QKV_CB), lambda t, c: (t, c)),
        out_shape=jax.ShapeDtypeStruct((S, QKVW), jnp.float32),
    )(x, Wqkv, ln1_w.reshape(1, D))
    q = qkv[:, :H * HD]
    k = qkv[:, H * HD:H * HD + KVH * HD]
    v = qkv[:, H * HD + KVH * HD:]

    def norm_rope(arr, w, nheads):
        return pl.pallas_call(
            _norm_rope_kernel,
            grid=(nheads, nt),
            in_specs=[
                pl.BlockSpec((BT, HD), lambda h, t: (t, h)),
                pl.BlockSpec((1, HD), lambda h, t: (0, 0)),
            ],
            out_specs=pl.BlockSpec((BT, HD), lambda h, t: (t, h)),
            out_shape=jax.ShapeDtypeStruct((S, nheads * HD), jnp.float32),
        )(arr, w.reshape(1, HD))

    q = norm_rope(q, q_norm_w, H)
    k = norm_rope(k, k_norm_w, KVH)

    rep = H // KVH
    o = pl.pallas_call(
        _attn_kernel,
        grid=(H, nt),
        in_specs=[
            pl.BlockSpec((BT, HD), lambda h, t: (t, h)),
            pl.BlockSpec((S, HD), lambda h, t: (0, h // rep)),
            pl.BlockSpec((S, HD), lambda h, t: (0, h // rep)),
        ],
        out_specs=pl.BlockSpec((BT, HD), lambda h, t: (t, h)),
        out_shape=jax.ShapeDtypeStruct((S, H * HD), jnp.float32),
    )(q, k, v)

    x1 = pl.pallas_call(
        _oproj_kernel,
        grid=(nt, D // QKV_CB),
        in_specs=[
            pl.BlockSpec((BT, H * HD), lambda t, c: (t, 0)),
            pl.BlockSpec((H * HD, QKV_CB), lambda t, c: (0, c)),
            pl.BlockSpec((BT, QKV_CB), lambda t, c: (t, c)),
        ],
        out_specs=pl.BlockSpec((BT, QKV_CB), lambda t, c: (t, c)),
        out_shape=jax.ShapeDtypeStruct((S, D), jnp.float32),
    )(o, Wo, x)

    h2, coef = pl.pallas_call(
        _router_kernel,
        grid=(nt,),
        in_specs=[
            pl.BlockSpec((BT, D), lambda t: (t, 0)),
            pl.BlockSpec((1, D), lambda t: (0, 0)),
            pl.BlockSpec((D, E), lambda t: (0, 0)),
        ],
        out_specs=[
            pl.BlockSpec((BT, D), lambda t: (t, 0)),
            pl.BlockSpec((BT, E), lambda t: (t, 0)),
        ],
        out_shape=[
            jax.ShapeDtypeStruct((S, D), jnp.float32),
            jax.ShapeDtypeStruct((S, E), jnp.float32),
        ],
    )(x1, ln2_w.reshape(1, D), router_W)

    out = pl.pallas_call(
        _moe_kernel,
        grid=(nt, E, F // FB),
        in_specs=[
            pl.BlockSpec((BT, D), lambda t, e, f: (t, 0)),
            pl.BlockSpec((1, D, FB), lambda t, e, f: (e, 0, f)),
            pl.BlockSpec((1, D, FB), lambda t, e, f: (e, 0, f)),
            pl.BlockSpec((1, FB, D), lambda t, e, f: (e, f, 0)),
            pl.BlockSpec((BT, E), lambda t, e, f: (t, 0)),
            pl.BlockSpec((BT, D), lambda t, e, f: (t, 0)),
        ],
        out_specs=pl.BlockSpec((BT, D), lambda t, e, f: (t, 0)),
        out_shape=jax.ShapeDtypeStruct((S, D), jnp.float32),
    )(h2, W_gate, W_up, W_down, coef, x1)

    return out.reshape(B, S, D)
